# baseline (speedup 1.0000x reference)
import functools

import jax
import jax.numpy as jnp
from jax import lax
from jax.experimental import pallas as pl
from jax.experimental.pallas import tpu as pltpu
from jax.experimental.pallas import tpu_sc as plsc

VOCAB = 100000
D = 64
BATCH = 4096
HIST = 50
NC, NS = 2, 16
NW = NC * NS
BAT_PER_W = BATCH // NW
FB = 8
NFILL = BAT_PER_W // FB
NGROUP = NFILL // 2


def _build():
    mesh = plsc.VectorSubcoreMesh(core_axis_name="c", subcore_axis_name="s")

    @functools.partial(
        pl.kernel,
        mesh=mesh,
        compiler_params=pltpu.CompilerParams(use_tc_tiling_on_sc=False),
        out_type=jax.ShapeDtypeStruct((BATCH, HIST, D), jnp.float32),
        scratch_types=[
            pltpu.VMEM((BAT_PER_W, HIST), jnp.int32),
            pltpu.VMEM((2 * FB, HIST, D), jnp.float32),
            pltpu.SemaphoreType.DMA,
            pltpu.SemaphoreType.DMA,
            pltpu.SemaphoreType.DMA,
            pltpu.SemaphoreType.DMA,
        ],
    )
    def emb_gather(idx_hbm, table_hbm, out_hbm, idx_v, rows_v,
                   gs0, gs1, os0, os1):
        wid = lax.axis_index("s") * NC + lax.axis_index("c")
        bat0 = wid * BAT_PER_W
        gsems = (gs0, gs1)
        osems = (os0, os1)

        pltpu.sync_copy(idx_hbm.at[pl.ds(bat0, BAT_PER_W)], idx_v)

        def issue_gathers(f, b):
            for k in range(FB):
                pltpu.async_copy(
                    table_hbm.at[idx_v.at[f * FB + k]],
                    rows_v.at[b * FB + k],
                    gsems[b])

        def wait_gathers(b):
            pltpu.make_async_copy(
                out_hbm.at[pl.ds(0, FB)],
                rows_v.at[pl.ds(b * FB, FB)], gsems[b]).wait()

        def issue_out(f, b):
            pltpu.async_copy(
                rows_v.at[pl.ds(b * FB, FB)],
                out_hbm.at[pl.ds(bat0 + f * FB, FB)],
                osems[b])

        def wait_out(b):
            pltpu.make_async_copy(
                rows_v.at[pl.ds(b * FB, FB)],
                out_hbm.at[pl.ds(bat0, FB)], osems[b]).wait()

        issue_gathers(0, 0)

        def group(g, carry):
            wait_gathers(0)
            issue_out(2 * g, 0)

            @pl.when(g > 0)
            def _():
                wait_out(1)
            issue_gathers(2 * g + 1, 1)

            wait_gathers(1)
            issue_out(2 * g + 1, 1)

            @pl.when(g < NGROUP - 1)
            def _():
                wait_out(0)
                issue_gathers(2 * g + 2, 0)
            return carry

        lax.fori_loop(0, NGROUP, group, 0)
        wait_out(0)
        wait_out(1)

    return emb_gather


_EMB_GATHER = _build()


def kernel(input_tokens, weight):
    return _EMB_GATHER(input_tokens.astype(jnp.int32), weight)
